# baseline (speedup 1.0000x reference)
import functools

import jax
import jax.numpy as jnp
from jax import lax
from jax.experimental import pallas as pl
from jax.experimental.pallas import tpu as pltpu
from jax.experimental.pallas import tpu_sc as plsc

B = 16384
D = 64
NC = 2
NS = 16
L = 16
NW = NC * NS
BPW = B // NW
CHUNK = 128
NCHUNK = BPW // CHUNK
NGROUP = BPW // L


def _sc_body(users_hbm, pos_hbm, neg_hbm, utab_hbm, itab_hbm,
             diff_hbm, sq_hbm,
             uidx, pidx, nidx, urows, prows, nrows, diff_v, sq_v, sem):
    wid = lax.axis_index("s") * NC + lax.axis_index("c")
    base = wid * BPW

    pltpu.sync_copy(users_hbm.at[pl.ds(wid * NCHUNK, NCHUNK)], uidx)
    pltpu.sync_copy(pos_hbm.at[pl.ds(wid * NCHUNK, NCHUNK)], pidx)
    pltpu.sync_copy(neg_hbm.at[pl.ds(wid * NCHUNK, NCHUNK)], nidx)

    urows2 = urows
    prows2 = prows
    nrows2 = nrows
    copies = []
    for c in range(NCHUNK):
        copies.append(pltpu.async_copy(
            utab_hbm.at[uidx.at[c]], urows2.at[pl.ds(c * CHUNK, CHUNK)], sem))
        copies.append(pltpu.async_copy(
            itab_hbm.at[pidx.at[c]], prows2.at[pl.ds(c * CHUNK, CHUNK)], sem))
        copies.append(pltpu.async_copy(
            itab_hbm.at[nidx.at[c]], nrows2.at[pl.ds(c * CHUNK, CHUNK)], sem))
    for cp in copies:
        cp.wait()

    lane = lax.iota(jnp.int32, L)
    zero = jnp.zeros((L,), jnp.float32)
    def group_body(g, sq_acc):
        rows = g * L + lane
        score = zero
        for d in range(D):
            col = jnp.full((L,), d, jnp.int32)
            u = plsc.load_gather(urows, [rows, col])
            p = plsc.load_gather(prows, [rows, col])
            n = plsc.load_gather(nrows, [rows, col])
            score = score + u * (p - n)
            sq_acc = sq_acc + (u * u + p * p + n * n)
        diff_v[pl.ds(g * L, L)] = score
        return sq_acc

    sq_acc = lax.fori_loop(0, NGROUP, group_body, zero)
    sq_v[...] = sq_acc

    pltpu.sync_copy(diff_v, diff_hbm.at[pl.ds(base, BPW)])
    pltpu.sync_copy(sq_v, sq_hbm.at[wid])


def _loss_body(diff_ref, sq_ref, out_ref):
    x = diff_ref[...]
    ls = jnp.minimum(x, 0.0) - jnp.log1p(jnp.exp(-jnp.abs(x)))
    loss = -jnp.sum(ls) / B
    reg = jnp.sum(sq_ref[...]) / B
    out_ref[...] = jnp.reshape(loss + 1e-5 * reg, (1, 1))


@jax.jit
def kernel(users, pos_items, neg_items, user_table, item_table):
    mesh = plsc.VectorSubcoreMesh(core_axis_name="c", subcore_axis_name="s")
    sc_fn = pl.kernel(
        _sc_body,
        out_type=[
            jax.ShapeDtypeStruct((B,), jnp.float32),
            jax.ShapeDtypeStruct((NW, L), jnp.float32),
        ],
        mesh=mesh,
        scratch_types=[
            pltpu.VMEM((NCHUNK, CHUNK), jnp.int32),
            pltpu.VMEM((NCHUNK, CHUNK), jnp.int32),
            pltpu.VMEM((NCHUNK, CHUNK), jnp.int32),
            pltpu.VMEM((BPW, D), jnp.float32),
            pltpu.VMEM((BPW, D), jnp.float32),
            pltpu.VMEM((BPW, D), jnp.float32),
            pltpu.VMEM((BPW,), jnp.float32),
            pltpu.VMEM((L,), jnp.float32),
            pltpu.SemaphoreType.DMA,
        ],
        compiler_params=pltpu.CompilerParams(
            needs_layout_passes=False, use_tc_tiling_on_sc=False),
    )
    diff, sq = sc_fn(
        users.reshape(B // CHUNK, CHUNK).astype(jnp.int32),
        pos_items.reshape(B // CHUNK, CHUNK).astype(jnp.int32),
        neg_items.reshape(B // CHUNK, CHUNK).astype(jnp.int32),
        user_table, item_table)

    out = pl.pallas_call(
        _loss_body,
        out_shape=jax.ShapeDtypeStruct((1, 1), jnp.float32),
    )(diff.reshape(128, 128), sq)
    return out[0, 0]
